# baseline (speedup 1.0000x reference)
import jax
import jax.numpy as jnp
from jax import lax
from jax.experimental import pallas as pl
from jax.experimental.pallas import tpu as pltpu
from jax.experimental.pallas import tpu_sc as plsc

N = 10000
E = 320000
D = 128
NPAD = 10240
NC = 2
NS = 16
NW = NC * NS
EPW = E // NW
CH = 80
NCHUNK = EPW // CH
GR = CH // 16
ROWS_PT = NPAD // NS
TAU = 1.0 / (128.0 ** 0.5)

_f32 = jnp.float32
_i32 = jnp.int32


def _proj_body(z_ref, wq_ref, bq_ref, wk_ref, bk_ref, wv_ref, bv_ref,
               q_ref, k_ref, v_ref):
    z = z_ref[...]
    q_ref[...] = jnp.dot(z, wq_ref[...], preferred_element_type=_f32) + bq_ref[...]
    k_ref[...] = jnp.dot(z, wk_ref[...], preferred_element_type=_f32) + bk_ref[...]
    v_ref[...] = jnp.dot(z, wv_ref[...], preferred_element_type=_f32) + bv_ref[...]


def _project(z, wqt, bq, wkt, bk, wvt, bv):
    blk = 1000
    grid = (N // blk,)
    zspec = pl.BlockSpec((blk, D), lambda i: (i, 0))
    wspec = pl.BlockSpec((D, D), lambda i: (0, 0))
    bspec = pl.BlockSpec((1, D), lambda i: (0, 0))
    ospec = pl.BlockSpec((blk, D), lambda i: (i, 0))
    out = pl.pallas_call(
        _proj_body,
        grid=grid,
        in_specs=[zspec, wspec, bspec, wspec, bspec, wspec, bspec],
        out_specs=[ospec, ospec, ospec],
        out_shape=[jax.ShapeDtypeStruct((N, D), _f32)] * 3,
    )(z, wqt, bq, wkt, bk, wvt, bv)
    return out


def _iota16():
    return lax.iota(_i32, 16)


def _edge_body(k_hbm, q_hbm, v_hbm, src_hbm, dst_hbm, zrow_hbm, zd_hbm,
               expe_hbm, hu_hbm, dp_hbm,
               src0, dst0, src1, dst1, dsc0, dsc1, k_rows, q_rows, v_rows,
               e0, e1, hu_sh, d_sh, sidx, sk, sq, sv, s_hu, s_d, s_e):
    c = lax.axis_index("c")
    s = lax.axis_index("s")
    wid = s * NC + c
    ebase = wid * EPW

    pltpu.sync_copy(zrow_hbm, hu_sh.at[pl.ds(s * ROWS_PT, ROWS_PT)])
    pltpu.sync_copy(zd_hbm, d_sh.at[pl.ds(s * ROWS_PT, ROWS_PT)])
    plsc.subcore_barrier()

    srcs = (src0, src1)
    dsts = (dst0, dst1)
    dscs = (dsc0, dsc1)
    ebufs = (e0, e1)

    iota16 = _iota16()

    def compute_dot(eb):
        def group_body(g, carry):
            def edge_body(r, evec):
                e = g * 16 + r
                acc = k_rows[e, pl.ds(0, 16)] * q_rows[e, pl.ds(0, 16)]
                for cc in range(1, D // 16):
                    sl = pl.ds(cc * 16, 16)
                    acc = acc + k_rows[e, sl] * q_rows[e, sl]
                se = jnp.sum(acc)
                return jnp.where(iota16 == r, se, evec)

            evec = lax.fori_loop(0, 16, edge_body, jnp.zeros((16,), _f32),
                                 unroll=4)
            eb[pl.ds(g * 16, 16)] = jnp.exp(evec * TAU)
            return carry

        lax.fori_loop(0, GR, group_body, 0)

    def scale_v(vb, eb):
        def group_body(g, carry):
            ev = eb[pl.ds(g * 16, 16)]

            def edge_body(r, carry2):
                e = g * 16 + r
                ee = jnp.sum(jnp.where(iota16 == r, ev, 0.0))
                for cc in range(D // 16):
                    sl = pl.ds(cc * 16, 16)
                    vb[e, sl] = vb[e, sl] * ee
                return carry2

            lax.fori_loop(0, 16, edge_body, 0, unroll=4)
            return carry

        lax.fori_loop(0, GR, group_body, 0)

    def chunk(g, p, first):
        sp, dp_ = srcs[p], dsts[p]
        sn, dn = srcs[1 - p], dsts[1 - p]
        ep, en = ebufs[p], ebufs[1 - p]
        pltpu.make_async_copy(k_hbm.at[sp], k_rows, sk).wait()
        pltpu.make_async_copy(q_hbm.at[dp_], q_rows, sq).wait()
        basen = jnp.minimum((g + 1) * CH + ebase, E - CH)
        ci_s = pltpu.async_copy(src_hbm.at[pl.ds(basen, CH)], sn, sidx)
        ci_d = pltpu.async_copy(dst_hbm.at[pl.ds(basen, CH)], dn, sidx)
        if not first:
            pltpu.make_async_copy(v_rows, hu_sh.at[dscs[1 - p]], s_hu).wait()
            pltpu.make_async_copy(en, d_sh.at[dscs[1 - p]], s_d).wait()
            pltpu.make_async_copy(
                en, expe_hbm.at[pl.ds(g * CH + ebase - CH, CH)], s_e).wait()
        cv = pltpu.async_copy(v_hbm.at[sp], v_rows, sv)
        compute_dot(ep)
        ci_s.wait()
        ci_d.wait()
        pltpu.async_copy(k_hbm.at[sn], k_rows, sk)
        pltpu.async_copy(q_hbm.at[dn], q_rows, sq)
        cv.wait()
        scale_v(v_rows, ep)
        for gg in range(GR):
            sl = pl.ds(gg * 16, 16)
            dscs[p][sl] = dp_[sl]
        pltpu.async_copy(v_rows, hu_sh.at[dscs[p]], s_hu, add=True)
        pltpu.async_copy(ep, d_sh.at[dscs[p]], s_d, add=True)
        pltpu.async_copy(ep, expe_hbm.at[pl.ds(g * CH + ebase, CH)], s_e)

    pltpu.sync_copy(src_hbm.at[pl.ds(ebase, CH)], src0)
    pltpu.sync_copy(dst_hbm.at[pl.ds(ebase, CH)], dst0)
    pltpu.async_copy(k_hbm.at[src0], k_rows, sk)
    pltpu.async_copy(q_hbm.at[dst0], q_rows, sq)
    chunk(jnp.int32(0), 0, True)

    def loop_body(it, carry):
        g1 = 1 + 2 * it
        chunk(g1, 1, False)
        chunk(g1 + 1, 0, False)
        return carry

    lax.fori_loop(0, (NCHUNK - 1) // 2, loop_body, 0)

    pltpu.make_async_copy(v_rows, hu_sh.at[dsc0], s_hu).wait()
    pltpu.make_async_copy(e0, d_sh.at[dsc0], s_d).wait()
    pltpu.make_async_copy(
        e0, expe_hbm.at[pl.ds(ebase + EPW - CH, CH)], s_e).wait()
    pltpu.make_async_copy(k_hbm.at[src1], k_rows, sk).wait()
    pltpu.make_async_copy(q_hbm.at[dst1], q_rows, sq).wait()
    plsc.subcore_barrier()

    pltpu.sync_copy(hu_sh.at[pl.ds(s * ROWS_PT, ROWS_PT)],
                    hu_hbm.at[c, pl.ds(s * ROWS_PT, ROWS_PT)])
    pltpu.sync_copy(d_sh.at[pl.ds(s * ROWS_PT, ROWS_PT)],
                    dp_hbm.at[pl.ds(c * NPAD + s * ROWS_PT, ROWS_PT)])


def _edge_phase(k, q, v, src, dst):
    zrow = jnp.zeros((ROWS_PT, D), _f32)
    zd = jnp.zeros((ROWS_PT,), _f32)
    mesh = plsc.VectorSubcoreMesh(core_axis_name="c", subcore_axis_name="s")
    fn = pl.kernel(
        _edge_body,
        out_type=[
            jax.ShapeDtypeStruct((E,), _f32),
            jax.ShapeDtypeStruct((NC, NPAD, D), _f32),
            jax.ShapeDtypeStruct((NC * NPAD,), _f32),
        ],
        mesh=mesh,
        compiler_params=pltpu.CompilerParams(needs_layout_passes=False),
        scratch_types=[
            pltpu.VMEM((CH,), _i32),
            pltpu.VMEM((CH,), _i32),
            pltpu.VMEM((CH,), _i32),
            pltpu.VMEM((CH,), _i32),
            pltpu.VMEM((CH,), _i32),
            pltpu.VMEM((CH,), _i32),
            pltpu.VMEM((CH, D), _f32),
            pltpu.VMEM((CH, D), _f32),
            pltpu.VMEM((CH, D), _f32),
            pltpu.VMEM((CH,), _f32),
            pltpu.VMEM((CH,), _f32),
            pltpu.VMEM_SHARED((NPAD, D), _f32),
            pltpu.VMEM_SHARED((NPAD,), _f32),
            pltpu.SemaphoreType.DMA,
            pltpu.SemaphoreType.DMA,
            pltpu.SemaphoreType.DMA,
            pltpu.SemaphoreType.DMA,
            pltpu.SemaphoreType.DMA,
            pltpu.SemaphoreType.DMA,
            pltpu.SemaphoreType.DMA,
        ],
    )
    return fn(k, q, v, src, dst, zrow, zd)


_HB = 80
_NHB = N // _HB
_CHB = 2000
_NCHB = EPW // _CHB


def _final_body(hu0_hbm, hu1_hbm, dp_hbm, expe_hbm, dst_hbm,
                h_hbm, alpha_hbm,
                dfbuf, dall, hu0, hu1, hout, ev_buf, dste_buf, alpha_buf,
                sem0, sem1):
    c = lax.axis_index("c")
    s = lax.axis_index("s")
    wid = s * NC + c
    iota16 = _iota16()

    pltpu.sync_copy(dp_hbm, dfbuf)

    def dred_body(i, carry):
        sl = pl.ds(i * 16, 16)
        dall[sl] = dfbuf[sl] + dfbuf[pl.ds(NPAD + i * 16, 16)]
        return carry

    lax.fori_loop(0, NPAD // 16, dred_body, 0, unroll=8)

    def h_body(t, carry):
        nb = (wid + t * NW) * _HB
        c0 = pltpu.async_copy(hu0_hbm.at[pl.ds(nb, _HB)], hu0, sem0)
        c1 = pltpu.async_copy(hu1_hbm.at[pl.ds(nb, _HB)], hu1, sem1)
        c0.wait()
        c1.wait()

        def grp_body(g2, carry2):
            dg = dall[pl.ds(nb + g2 * 16, 16)]
            rcp = jnp.where(dg > 0.0, 1.0 / dg, 0.0)

            def row_body(r2, carry3):
                row = g2 * 16 + r2
                ee = jnp.sum(jnp.where(iota16 == r2, rcp, 0.0))
                for cc in range(D // 16):
                    sl = pl.ds(cc * 16, 16)
                    hout[row, sl] = (hu0[row, sl] + hu1[row, sl]) * ee
                return carry3

            lax.fori_loop(0, 16, row_body, 0, unroll=4)
            return carry2

        lax.fori_loop(0, _HB // 16, grp_body, 0)
        pltpu.sync_copy(hout, h_hbm.at[pl.ds(nb, _HB)])
        return carry

    my_blocks = (_NHB - wid + NW - 1) // NW
    lax.fori_loop(0, my_blocks, h_body, 0)

    def a_body(ch, carry):
        base = wid * EPW + ch * _CHB
        ce = pltpu.async_copy(expe_hbm.at[pl.ds(base, _CHB)], ev_buf, sem0)
        cd = pltpu.async_copy(dst_hbm.at[pl.ds(base, _CHB)], dste_buf, sem1)
        ce.wait()
        cd.wait()

        def g_body(g, carry2):
            sl = pl.ds(g * 16, 16)
            dv = plsc.load_gather(dall, [dste_buf[sl]])
            alpha_buf[sl] = ev_buf[sl] / dv
            return carry2

        lax.fori_loop(0, _CHB // 16, g_body, 0, unroll=4)
        pltpu.sync_copy(alpha_buf, alpha_hbm.at[pl.ds(base, _CHB)])
        return carry

    lax.fori_loop(0, _NCHB, a_body, 0)


def _finalize(hu_p, dp, expe, dst):
    mesh = plsc.VectorSubcoreMesh(core_axis_name="c", subcore_axis_name="s")
    fn = pl.kernel(
        _final_body,
        out_type=[
            jax.ShapeDtypeStruct((N, D), _f32),
            jax.ShapeDtypeStruct((E,), _f32),
        ],
        mesh=mesh,
        compiler_params=pltpu.CompilerParams(needs_layout_passes=False),
        scratch_types=[
            pltpu.VMEM((NC * NPAD,), _f32),
            pltpu.VMEM((NPAD,), _f32),
            pltpu.VMEM((_HB, D), _f32),
            pltpu.VMEM((_HB, D), _f32),
            pltpu.VMEM((_HB, D), _f32),
            pltpu.VMEM((_CHB,), _f32),
            pltpu.VMEM((_CHB,), _i32),
            pltpu.VMEM((_CHB,), _f32),
            pltpu.SemaphoreType.DMA,
            pltpu.SemaphoreType.DMA,
        ],
    )
    return fn(hu_p[0], hu_p[1], dp, expe, dst)


def kernel(z, edge_index, Wq, bq, Wk, bk, Wv, bv):
    q, k, v = _project(z, Wq.T, bq.reshape(1, D), Wk.T, bk.reshape(1, D),
                       Wv.T, bv.reshape(1, D))
    src = edge_index[0]
    dst = edge_index[1]
    expe, hu_p, dp = _edge_phase(k, q, v, src, dst)
    h, alpha = _finalize(hu_p, dp, expe, dst)
    return h, alpha


# speedup vs baseline: 1.0073x; 1.0073x over previous
import jax
import jax.numpy as jnp
from jax import lax
from jax.experimental import pallas as pl
from jax.experimental.pallas import tpu as pltpu
from jax.experimental.pallas import tpu_sc as plsc

N = 10000
E = 320000
D = 128
NPAD = 10240
NC = 2
NS = 16
NW = NC * NS
EPW = E // NW
CH = 80
NCHUNK = EPW // CH
GR = CH // 16
ROWS_PT = NPAD // NS
TAU = 1.0 / (128.0 ** 0.5)

_f32 = jnp.float32
_i32 = jnp.int32


def _proj_body(z_ref, wq_ref, bq_ref, wk_ref, bk_ref, wv_ref, bv_ref,
               q_ref, k_ref, v_ref):
    z = z_ref[...]
    q_ref[...] = jnp.dot(z, wq_ref[...], preferred_element_type=_f32) + bq_ref[...]
    k_ref[...] = jnp.dot(z, wk_ref[...], preferred_element_type=_f32) + bk_ref[...]
    v_ref[...] = jnp.dot(z, wv_ref[...], preferred_element_type=_f32) + bv_ref[...]


def _project(z, wqt, bq, wkt, bk, wvt, bv):
    blk = 1000
    grid = (N // blk,)
    zspec = pl.BlockSpec((blk, D), lambda i: (i, 0))
    wspec = pl.BlockSpec((D, D), lambda i: (0, 0))
    bspec = pl.BlockSpec((1, D), lambda i: (0, 0))
    ospec = pl.BlockSpec((blk, D), lambda i: (i, 0))
    out = pl.pallas_call(
        _proj_body,
        grid=grid,
        in_specs=[zspec, wspec, bspec, wspec, bspec, wspec, bspec],
        out_specs=[ospec, ospec, ospec],
        out_shape=[jax.ShapeDtypeStruct((N, D), _f32)] * 3,
    )(z, wqt, bq, wkt, bk, wvt, bv)
    return out


def _iota16():
    return lax.iota(_i32, 16)


def _edge_body(k_hbm, q_hbm, v_hbm, src_hbm, dst_hbm, zrow_hbm, zd_hbm,
               expe_hbm, hu_hbm, dp_hbm,
               src0, dst0, src1, dst1, dsc0, dsc1, k_rows, q_rows, v_rows,
               e0, e1, hu_sh, d_sh, sidx, sk, sq, sv, s_hu, s_d, s_e):
    c = lax.axis_index("c")
    s = lax.axis_index("s")
    wid = s * NC + c
    ebase = wid * EPW

    pltpu.sync_copy(zrow_hbm, hu_sh.at[pl.ds(s * ROWS_PT, ROWS_PT)])
    pltpu.sync_copy(zd_hbm, d_sh.at[pl.ds(s * ROWS_PT, ROWS_PT)])
    plsc.subcore_barrier()

    srcs = (src0, src1)
    dsts = (dst0, dst1)
    dscs = (dsc0, dsc1)
    ebufs = (e0, e1)

    iota16 = _iota16()

    def compute_dot(eb):
        def group_body(g, carry):
            def edge_body(r, evec):
                e = g * 16 + r
                acc = k_rows[e, pl.ds(0, 16)] * q_rows[e, pl.ds(0, 16)]
                for cc in range(1, D // 16):
                    sl = pl.ds(cc * 16, 16)
                    acc = acc + k_rows[e, sl] * q_rows[e, sl]
                se = jnp.sum(acc)
                return jnp.where(iota16 == r, se, evec)

            evec = lax.fori_loop(0, 16, edge_body, jnp.zeros((16,), _f32),
                                 unroll=4)
            eb[pl.ds(g * 16, 16)] = jnp.exp(evec * TAU)
            return carry

        lax.fori_loop(0, GR, group_body, 0)

    def scale_v(vb, eb):
        def group_body(g, carry):
            ev = eb[pl.ds(g * 16, 16)]

            def edge_body(r, carry2):
                e = g * 16 + r
                ee = jnp.sum(jnp.where(iota16 == r, ev, 0.0))
                for cc in range(D // 16):
                    sl = pl.ds(cc * 16, 16)
                    vb[e, sl] = vb[e, sl] * ee
                return carry2

            lax.fori_loop(0, 16, edge_body, 0, unroll=4)
            return carry

        lax.fori_loop(0, GR, group_body, 0)

    def chunk(g, p, first):
        sp, dp_ = srcs[p], dsts[p]
        sn, dn = srcs[1 - p], dsts[1 - p]
        ep, en = ebufs[p], ebufs[1 - p]
        pltpu.make_async_copy(k_hbm.at[sp], k_rows, sk).wait()
        pltpu.make_async_copy(q_hbm.at[dp_], q_rows, sq).wait()
        basen = jnp.minimum((g + 1) * CH + ebase, E - CH)
        ci_s = pltpu.async_copy(src_hbm.at[pl.ds(basen, CH)], sn, sidx)
        ci_d = pltpu.async_copy(dst_hbm.at[pl.ds(basen, CH)], dn, sidx)
        if not first:
            pltpu.make_async_copy(v_rows, hu_sh.at[dscs[1 - p]], s_hu).wait()
            pltpu.make_async_copy(en, d_sh.at[dscs[1 - p]], s_d).wait()
            pltpu.make_async_copy(
                en, expe_hbm.at[pl.ds(g * CH + ebase - CH, CH)], s_e).wait()
        cv = pltpu.async_copy(v_hbm.at[sp], v_rows, sv)
        compute_dot(ep)
        ci_s.wait()
        ci_d.wait()
        pltpu.async_copy(k_hbm.at[sn], k_rows, sk)
        pltpu.async_copy(q_hbm.at[dn], q_rows, sq)
        cv.wait()
        scale_v(v_rows, ep)
        for gg in range(GR):
            sl = pl.ds(gg * 16, 16)
            dscs[p][sl] = dp_[sl]
        pltpu.async_copy(v_rows, hu_sh.at[dscs[p]], s_hu, add=True)
        pltpu.async_copy(ep, d_sh.at[dscs[p]], s_d, add=True)
        pltpu.async_copy(ep, expe_hbm.at[pl.ds(g * CH + ebase, CH)], s_e)

    pltpu.sync_copy(src_hbm.at[pl.ds(ebase, CH)], src0)
    pltpu.sync_copy(dst_hbm.at[pl.ds(ebase, CH)], dst0)
    pltpu.async_copy(k_hbm.at[src0], k_rows, sk)
    pltpu.async_copy(q_hbm.at[dst0], q_rows, sq)
    chunk(jnp.int32(0), 0, True)

    def loop_body(it, carry):
        g1 = 1 + 2 * it
        chunk(g1, 1, False)
        chunk(g1 + 1, 0, False)
        return carry

    lax.fori_loop(0, (NCHUNK - 1) // 2, loop_body, 0)

    pltpu.make_async_copy(v_rows, hu_sh.at[dsc0], s_hu).wait()
    pltpu.make_async_copy(e0, d_sh.at[dsc0], s_d).wait()
    pltpu.make_async_copy(
        e0, expe_hbm.at[pl.ds(ebase + EPW - CH, CH)], s_e).wait()
    pltpu.make_async_copy(k_hbm.at[src1], k_rows, sk).wait()
    pltpu.make_async_copy(q_hbm.at[dst1], q_rows, sq).wait()
    plsc.subcore_barrier()

    pltpu.sync_copy(hu_sh.at[pl.ds(s * ROWS_PT, ROWS_PT)],
                    hu_hbm.at[c, pl.ds(s * ROWS_PT, ROWS_PT)])
    pltpu.sync_copy(d_sh.at[pl.ds(s * ROWS_PT, ROWS_PT)],
                    dp_hbm.at[pl.ds(c * NPAD + s * ROWS_PT, ROWS_PT)])


def _edge_phase(k, q, v, src, dst):
    zrow = jnp.zeros((ROWS_PT, D), _f32)
    zd = jnp.zeros((ROWS_PT,), _f32)
    mesh = plsc.VectorSubcoreMesh(core_axis_name="c", subcore_axis_name="s")
    fn = pl.kernel(
        _edge_body,
        out_type=[
            jax.ShapeDtypeStruct((E,), _f32),
            jax.ShapeDtypeStruct((NC, NPAD, D), _f32),
            jax.ShapeDtypeStruct((NC * NPAD,), _f32),
        ],
        mesh=mesh,
        compiler_params=pltpu.CompilerParams(needs_layout_passes=False),
        scratch_types=[
            pltpu.VMEM((CH,), _i32),
            pltpu.VMEM((CH,), _i32),
            pltpu.VMEM((CH,), _i32),
            pltpu.VMEM((CH,), _i32),
            pltpu.VMEM((CH,), _i32),
            pltpu.VMEM((CH,), _i32),
            pltpu.VMEM((CH, D), _f32),
            pltpu.VMEM((CH, D), _f32),
            pltpu.VMEM((CH, D), _f32),
            pltpu.VMEM((CH,), _f32),
            pltpu.VMEM((CH,), _f32),
            pltpu.VMEM_SHARED((NPAD, D), _f32),
            pltpu.VMEM_SHARED((NPAD,), _f32),
            pltpu.SemaphoreType.DMA,
            pltpu.SemaphoreType.DMA,
            pltpu.SemaphoreType.DMA,
            pltpu.SemaphoreType.DMA,
            pltpu.SemaphoreType.DMA,
            pltpu.SemaphoreType.DMA,
            pltpu.SemaphoreType.DMA,
        ],
    )
    return fn(k, q, v, src, dst, zrow, zd)


_HB = 80
_NHB = N // _HB
_CHB = 2000
_NCHB = EPW // _CHB


def _final_body(hu0_hbm, hu1_hbm, dp_hbm, expe_hbm, dst_hbm,
                h_hbm, alpha_hbm,
                dfbuf, dall, hu0, hu1, hout, ev_buf, dste_buf, alpha_buf,
                sem0, sem1, sem2, sem3):
    c = lax.axis_index("c")
    s = lax.axis_index("s")
    wid = s * NC + c
    iota16 = _iota16()
    ebase = wid * EPW

    ce = pltpu.async_copy(expe_hbm.at[pl.ds(ebase, EPW)], ev_buf, sem2)
    cd = pltpu.async_copy(dst_hbm.at[pl.ds(ebase, EPW)], dste_buf, sem3)

    pltpu.sync_copy(dp_hbm, dfbuf)

    def dred_body(i, carry):
        sl = pl.ds(i * 16, 16)
        dall[sl] = dfbuf[sl] + dfbuf[pl.ds(NPAD + i * 16, 16)]
        return carry

    lax.fori_loop(0, NPAD // 16, dred_body, 0, unroll=8)

    def h_body(t, carry):
        nb = (wid + t * NW) * _HB
        c0 = pltpu.async_copy(hu0_hbm.at[pl.ds(nb, _HB)], hu0, sem0)
        c1 = pltpu.async_copy(hu1_hbm.at[pl.ds(nb, _HB)], hu1, sem1)
        c0.wait()
        c1.wait()

        def grp_body(g2, carry2):
            dg = dall[pl.ds(nb + g2 * 16, 16)]
            rcp = jnp.where(dg > 0.0, 1.0 / dg, 0.0)

            def row_body(r2, carry3):
                row = g2 * 16 + r2
                ee = jnp.sum(jnp.where(iota16 == r2, rcp, 0.0))
                for cc in range(D // 16):
                    sl = pl.ds(cc * 16, 16)
                    hout[row, sl] = (hu0[row, sl] + hu1[row, sl]) * ee
                return carry3

            lax.fori_loop(0, 16, row_body, 0, unroll=4)
            return carry2

        lax.fori_loop(0, _HB // 16, grp_body, 0)
        pltpu.sync_copy(hout, h_hbm.at[pl.ds(nb, _HB)])
        return carry

    my_blocks = (_NHB - wid + NW - 1) // NW
    lax.fori_loop(0, my_blocks, h_body, 0)

    ce.wait()
    cd.wait()

    def g_body(g, carry2):
        sl = pl.ds(g * 16, 16)
        dv = plsc.load_gather(dall, [dste_buf[sl]])
        alpha_buf[sl] = ev_buf[sl] / dv
        return carry2

    lax.fori_loop(0, EPW // 16, g_body, 0, unroll=4)
    pltpu.sync_copy(alpha_buf, alpha_hbm.at[pl.ds(ebase, EPW)])


def _finalize(hu_p, dp, expe, dst):
    mesh = plsc.VectorSubcoreMesh(core_axis_name="c", subcore_axis_name="s")
    fn = pl.kernel(
        _final_body,
        out_type=[
            jax.ShapeDtypeStruct((N, D), _f32),
            jax.ShapeDtypeStruct((E,), _f32),
        ],
        mesh=mesh,
        compiler_params=pltpu.CompilerParams(needs_layout_passes=False),
        scratch_types=[
            pltpu.VMEM((NC * NPAD,), _f32),
            pltpu.VMEM((NPAD,), _f32),
            pltpu.VMEM((_HB, D), _f32),
            pltpu.VMEM((_HB, D), _f32),
            pltpu.VMEM((_HB, D), _f32),
            pltpu.VMEM((EPW,), _f32),
            pltpu.VMEM((EPW,), _i32),
            pltpu.VMEM((EPW,), _f32),
            pltpu.SemaphoreType.DMA,
            pltpu.SemaphoreType.DMA,
            pltpu.SemaphoreType.DMA,
            pltpu.SemaphoreType.DMA,
        ],
    )
    return fn(hu_p[0], hu_p[1], dp, expe, dst)


def kernel(z, edge_index, Wq, bq, Wk, bk, Wv, bv):
    q, k, v = _project(z, Wq.T, bq.reshape(1, D), Wk.T, bk.reshape(1, D),
                       Wv.T, bv.reshape(1, D))
    src = edge_index[0]
    dst = edge_index[1]
    expe, hu_p, dp = _edge_phase(k, q, v, src, dst)
    h, alpha = _finalize(hu_p, dp, expe, dst)
    return h, alpha


# speedup vs baseline: 1.0219x; 1.0146x over previous
import jax
import jax.numpy as jnp
from jax import lax
from jax.experimental import pallas as pl
from jax.experimental.pallas import tpu as pltpu
from jax.experimental.pallas import tpu_sc as plsc

N = 10000
E = 320000
D = 128
NPAD = 10240
NC = 2
NS = 16
NW = NC * NS
EPW = E // NW
CH = 80
NCHUNK = EPW // CH
GR = CH // 16
ROWS_PT = NPAD // NS
TAU = 1.0 / (128.0 ** 0.5)

_f32 = jnp.float32
_i32 = jnp.int32


def _proj_body(z_ref, wq_ref, bq_ref, wk_ref, bk_ref, wv_ref, bv_ref,
               q_ref, k_ref, v_ref):
    z = z_ref[...]
    q_ref[...] = jnp.dot(z, wq_ref[...], preferred_element_type=_f32) + bq_ref[...]
    k_ref[...] = jnp.dot(z, wk_ref[...], preferred_element_type=_f32) + bk_ref[...]
    v_ref[...] = jnp.dot(z, wv_ref[...], preferred_element_type=_f32) + bv_ref[...]


def _project(z, wqt, bq, wkt, bk, wvt, bv):
    blk = 1000
    grid = (N // blk,)
    zspec = pl.BlockSpec((blk, D), lambda i: (i, 0))
    wspec = pl.BlockSpec((D, D), lambda i: (0, 0))
    bspec = pl.BlockSpec((1, D), lambda i: (0, 0))
    ospec = pl.BlockSpec((blk, D), lambda i: (i, 0))
    out = pl.pallas_call(
        _proj_body,
        grid=grid,
        in_specs=[zspec, wspec, bspec, wspec, bspec, wspec, bspec],
        out_specs=[ospec, ospec, ospec],
        out_shape=[jax.ShapeDtypeStruct((N, D), _f32)] * 3,
    )(z, wqt, bq, wkt, bk, wvt, bv)
    return out


def _iota16():
    return lax.iota(_i32, 16)


def _edge_body(k_hbm, q_hbm, v_hbm, src_hbm, dst_hbm, zrow_hbm, zd_hbm,
               expe_hbm, hu_hbm, dp_hbm,
               src0, dst0, src1, dst1, dsc0, dsc1, k_rows, q_rows, v_rows,
               e0, e1, hu_sh, d_sh, sidx, sk, sq, sv, s_hu, s_d, s_e):
    c = lax.axis_index("c")
    s = lax.axis_index("s")
    wid = s * NC + c
    ebase = wid * EPW

    pltpu.sync_copy(zrow_hbm, hu_sh.at[pl.ds(s * ROWS_PT, ROWS_PT)])
    pltpu.sync_copy(zd_hbm, d_sh.at[pl.ds(s * ROWS_PT, ROWS_PT)])
    plsc.subcore_barrier()

    srcs = (src0, src1)
    dsts = (dst0, dst1)
    dscs = (dsc0, dsc1)
    ebufs = (e0, e1)

    iota16 = _iota16()

    def compute_dot(eb):
        def group_body(g, carry):
            def edge_body(r, evec):
                e = g * 16 + r
                acc = k_rows[e, pl.ds(0, 16)] * q_rows[e, pl.ds(0, 16)]
                for cc in range(1, D // 16):
                    sl = pl.ds(cc * 16, 16)
                    acc = acc + k_rows[e, sl] * q_rows[e, sl]
                se = jnp.sum(acc)
                return jnp.where(iota16 == r, se, evec)

            evec = lax.fori_loop(0, 16, edge_body, jnp.zeros((16,), _f32),
                                 unroll=4)
            eb[pl.ds(g * 16, 16)] = jnp.exp(evec * TAU)
            return carry

        lax.fori_loop(0, GR, group_body, 0)

    def scale_v(vb, eb):
        def group_body(g, carry):
            ev = eb[pl.ds(g * 16, 16)]

            def edge_body(r, carry2):
                e = g * 16 + r
                ee = jnp.sum(jnp.where(iota16 == r, ev, 0.0))
                for cc in range(D // 16):
                    sl = pl.ds(cc * 16, 16)
                    vb[e, sl] = vb[e, sl] * ee
                return carry2

            lax.fori_loop(0, 16, edge_body, 0, unroll=4)
            return carry

        lax.fori_loop(0, GR, group_body, 0)

    def chunk(g, p, first):
        sp, dp_ = srcs[p], dsts[p]
        sn, dn = srcs[1 - p], dsts[1 - p]
        ep, en = ebufs[p], ebufs[1 - p]
        pltpu.make_async_copy(k_hbm.at[sp], k_rows, sk).wait()
        pltpu.make_async_copy(q_hbm.at[dp_], q_rows, sq).wait()
        basen = jnp.minimum((g + 1) * CH + ebase, E - CH)
        ci_s = pltpu.async_copy(src_hbm.at[pl.ds(basen, CH)], sn, sidx)
        ci_d = pltpu.async_copy(dst_hbm.at[pl.ds(basen, CH)], dn, sidx)
        if not first:
            pltpu.make_async_copy(v_rows, hu_sh.at[dscs[1 - p]], s_hu).wait()
            pltpu.make_async_copy(en, d_sh.at[dscs[1 - p]], s_d).wait()
            pltpu.make_async_copy(
                en, expe_hbm.at[pl.ds(g * CH + ebase - CH, CH)], s_e).wait()
        cv = pltpu.async_copy(v_hbm.at[sp], v_rows, sv)
        compute_dot(ep)
        ci_s.wait()
        ci_d.wait()
        pltpu.async_copy(k_hbm.at[sn], k_rows, sk)
        pltpu.async_copy(q_hbm.at[dn], q_rows, sq)
        cv.wait()
        scale_v(v_rows, ep)
        for gg in range(GR):
            sl = pl.ds(gg * 16, 16)
            dscs[p][sl] = dp_[sl]
        pltpu.async_copy(v_rows, hu_sh.at[dscs[p]], s_hu, add=True)
        pltpu.async_copy(ep, d_sh.at[dscs[p]], s_d, add=True)
        pltpu.async_copy(ep, expe_hbm.at[pl.ds(g * CH + ebase, CH)], s_e)

    pltpu.sync_copy(src_hbm.at[pl.ds(ebase, CH)], src0)
    pltpu.sync_copy(dst_hbm.at[pl.ds(ebase, CH)], dst0)
    pltpu.async_copy(k_hbm.at[src0], k_rows, sk)
    pltpu.async_copy(q_hbm.at[dst0], q_rows, sq)
    chunk(jnp.int32(0), 0, True)

    def loop_body(it, carry):
        g1 = 1 + 2 * it
        chunk(g1, 1, False)
        chunk(g1 + 1, 0, False)
        return carry

    lax.fori_loop(0, (NCHUNK - 1) // 2, loop_body, 0)

    pltpu.make_async_copy(v_rows, hu_sh.at[dsc0], s_hu).wait()
    pltpu.make_async_copy(e0, d_sh.at[dsc0], s_d).wait()
    pltpu.make_async_copy(
        e0, expe_hbm.at[pl.ds(ebase + EPW - CH, CH)], s_e).wait()
    pltpu.make_async_copy(k_hbm.at[src1], k_rows, sk).wait()
    pltpu.make_async_copy(q_hbm.at[dst1], q_rows, sq).wait()
    plsc.subcore_barrier()

    pltpu.sync_copy(hu_sh.at[pl.ds(s * ROWS_PT, ROWS_PT)],
                    hu_hbm.at[pl.ds(c * NPAD + s * ROWS_PT, ROWS_PT)])
    pltpu.sync_copy(d_sh.at[pl.ds(s * ROWS_PT, ROWS_PT)],
                    dp_hbm.at[pl.ds(c * NPAD + s * ROWS_PT, ROWS_PT)])


def _edge_phase(k, q, v, src, dst):
    zrow = jnp.zeros((ROWS_PT, D), _f32)
    zd = jnp.zeros((ROWS_PT,), _f32)
    mesh = plsc.VectorSubcoreMesh(core_axis_name="c", subcore_axis_name="s")
    fn = pl.kernel(
        _edge_body,
        out_type=[
            jax.ShapeDtypeStruct((E,), _f32),
            jax.ShapeDtypeStruct((NC * NPAD, D), _f32),
            jax.ShapeDtypeStruct((NC * NPAD,), _f32),
        ],
        mesh=mesh,
        compiler_params=pltpu.CompilerParams(needs_layout_passes=False),
        scratch_types=[
            pltpu.VMEM((CH,), _i32),
            pltpu.VMEM((CH,), _i32),
            pltpu.VMEM((CH,), _i32),
            pltpu.VMEM((CH,), _i32),
            pltpu.VMEM((CH,), _i32),
            pltpu.VMEM((CH,), _i32),
            pltpu.VMEM((CH, D), _f32),
            pltpu.VMEM((CH, D), _f32),
            pltpu.VMEM((CH, D), _f32),
            pltpu.VMEM((CH,), _f32),
            pltpu.VMEM((CH,), _f32),
            pltpu.VMEM_SHARED((NPAD, D), _f32),
            pltpu.VMEM_SHARED((NPAD,), _f32),
            pltpu.SemaphoreType.DMA,
            pltpu.SemaphoreType.DMA,
            pltpu.SemaphoreType.DMA,
            pltpu.SemaphoreType.DMA,
            pltpu.SemaphoreType.DMA,
            pltpu.SemaphoreType.DMA,
            pltpu.SemaphoreType.DMA,
        ],
    )
    return fn(k, q, v, src, dst, zrow, zd)


_HB = 80
_NHB = N // _HB
_CHB = 2000
_NCHB = EPW // _CHB


def _final_body(hu_hbm, dp_hbm, expe_hbm, dst_hbm,
                h_hbm, alpha_hbm,
                dfbuf, dall, hu0, hu1, hout, ev_buf, dste_buf, alpha_buf,
                sem0, sem1, sem2, sem3):
    c = lax.axis_index("c")
    s = lax.axis_index("s")
    wid = s * NC + c
    iota16 = _iota16()
    ebase = wid * EPW

    ce = pltpu.async_copy(expe_hbm.at[pl.ds(ebase, EPW)], ev_buf, sem2)
    cd = pltpu.async_copy(dst_hbm.at[pl.ds(ebase, EPW)], dste_buf, sem3)

    pltpu.sync_copy(dp_hbm, dfbuf)

    def dred_body(i, carry):
        sl = pl.ds(i * 16, 16)
        dall[sl] = dfbuf[sl] + dfbuf[pl.ds(NPAD + i * 16, 16)]
        return carry

    lax.fori_loop(0, NPAD // 16, dred_body, 0, unroll=8)

    def h_body(t, carry):
        nb = (wid + t * NW) * _HB
        c0 = pltpu.async_copy(hu_hbm.at[pl.ds(nb, _HB)], hu0, sem0)
        c1 = pltpu.async_copy(hu_hbm.at[pl.ds(NPAD + nb, _HB)], hu1, sem1)
        c0.wait()
        c1.wait()

        def grp_body(g2, carry2):
            dg = dall[pl.ds(nb + g2 * 16, 16)]
            rcp = jnp.where(dg > 0.0, 1.0 / dg, 0.0)

            def row_body(r2, carry3):
                row = g2 * 16 + r2
                ee = jnp.sum(jnp.where(iota16 == r2, rcp, 0.0))
                for cc in range(D // 16):
                    sl = pl.ds(cc * 16, 16)
                    hout[row, sl] = (hu0[row, sl] + hu1[row, sl]) * ee
                return carry3

            lax.fori_loop(0, 16, row_body, 0, unroll=4)
            return carry2

        lax.fori_loop(0, _HB // 16, grp_body, 0)
        pltpu.sync_copy(hout, h_hbm.at[pl.ds(nb, _HB)])
        return carry

    my_blocks = (_NHB - wid + NW - 1) // NW
    lax.fori_loop(0, my_blocks, h_body, 0)

    ce.wait()
    cd.wait()

    def g_body(g, carry2):
        sl = pl.ds(g * 16, 16)
        dv = plsc.load_gather(dall, [dste_buf[sl]])
        alpha_buf[sl] = ev_buf[sl] / dv
        return carry2

    lax.fori_loop(0, EPW // 16, g_body, 0, unroll=4)
    pltpu.sync_copy(alpha_buf, alpha_hbm.at[pl.ds(ebase, EPW)])


def _finalize(hu_p, dp, expe, dst):
    mesh = plsc.VectorSubcoreMesh(core_axis_name="c", subcore_axis_name="s")
    fn = pl.kernel(
        _final_body,
        out_type=[
            jax.ShapeDtypeStruct((N, D), _f32),
            jax.ShapeDtypeStruct((E,), _f32),
        ],
        mesh=mesh,
        compiler_params=pltpu.CompilerParams(needs_layout_passes=False),
        scratch_types=[
            pltpu.VMEM((NC * NPAD,), _f32),
            pltpu.VMEM((NPAD,), _f32),
            pltpu.VMEM((_HB, D), _f32),
            pltpu.VMEM((_HB, D), _f32),
            pltpu.VMEM((_HB, D), _f32),
            pltpu.VMEM((EPW,), _f32),
            pltpu.VMEM((EPW,), _i32),
            pltpu.VMEM((EPW,), _f32),
            pltpu.SemaphoreType.DMA,
            pltpu.SemaphoreType.DMA,
            pltpu.SemaphoreType.DMA,
            pltpu.SemaphoreType.DMA,
        ],
    )
    return fn(hu_p, dp, expe, dst)


def kernel(z, edge_index, Wq, bq, Wk, bk, Wv, bv):
    q, k, v = _project(z, Wq.T, bq.reshape(1, D), Wk.T, bk.reshape(1, D),
                       Wv.T, bv.reshape(1, D))
    src = edge_index[0]
    dst = edge_index[1]
    expe, hu_p, dp = _edge_phase(k, q, v, src, dst)
    h, alpha = _finalize(hu_p, dp, expe, dst)
    return h, alpha
